# Initial kernel scaffold; baseline (speedup 1.0000x reference)
import jax
import jax.numpy as jnp
from jax.experimental import pallas as pl


def kernel(x, edge_index, edge_type, node_type, node_type_emb, W1, root1, b1, g1, be1, W2, root2, b2, g2, be2):
    raise NotImplementedError("write your pallas kernel here")


# speedup vs baseline: 18.6473x; 18.6473x over previous
import functools

import jax
import jax.numpy as jnp
from jax import lax
from jax.experimental import pallas as pl
from jax.experimental.pallas import tpu as pltpu
from jax.experimental.pallas import tpu_sc as plsc

NC = 2
NS = 16
NW = NC * NS

_SC_MESH = dict(core_axis_name="c", subcore_axis_name="s",
                num_cores=NC, num_subcores=NS)


def _ln(h, g, b):
    mu = jnp.mean(h, axis=-1, keepdims=True)
    var = jnp.mean((h - mu) ** 2, axis=-1, keepdims=True)
    return (h - mu) / jnp.sqrt(var + 1e-5) * g + b


def _gelu(h):
    return 0.5 * h * (1.0 + lax.erf(h * 0.7071067811865476))


def _tc_first(x, nt2, emb, W, root, b1r, BN):
    N, D = x.shape
    R, _, H = W.shape
    NB = N // BN
    T = emb.shape[0]

    def body(x_ref, nt_ref, emb_ref, w_ref, root_ref, b_ref, h_ref, o_ref):
        r = pl.program_id(1)
        nt = nt_ref[...]
        onehot = (nt == lax.broadcasted_iota(jnp.int32, (BN, T), 1)
                  ).astype(jnp.float32)
        typed = x_ref[...] + jnp.dot(onehot, emb_ref[...],
                                     preferred_element_type=jnp.float32)
        h_ref[0] = jnp.dot(typed, w_ref[0],
                           preferred_element_type=jnp.float32)

        @pl.when(r == 0)
        def _():
            o_ref[...] = jnp.dot(typed, root_ref[...],
                                 preferred_element_type=jnp.float32) \
                         + b_ref[...]

    return pl.pallas_call(
        body,
        grid=(NB, R),
        in_specs=[
            pl.BlockSpec((BN, D), lambda i, r: (i, 0)),
            pl.BlockSpec((BN, 1), lambda i, r: (i, 0)),
            pl.BlockSpec((T, D), lambda i, r: (0, 0)),
            pl.BlockSpec((1, D, H), lambda i, r: (r, 0, 0)),
            pl.BlockSpec((D, H), lambda i, r: (0, 0)),
            pl.BlockSpec((1, H), lambda i, r: (0, 0)),
        ],
        out_specs=[
            pl.BlockSpec((1, BN, H), lambda i, r: (r, i, 0)),
            pl.BlockSpec((BN, H), lambda i, r: (i, 0)),
        ],
        out_shape=[
            jax.ShapeDtypeStruct((R, N, H), jnp.float32),
            jax.ShapeDtypeStruct((N, H), jnp.float32),
        ],
    )(x, nt2, emb, W, root, b1r)


def _tc_mid(out0, agg, g1r, be1r, W, root, b2r, BN):
    N, H = out0.shape
    R, _, D = W.shape
    NB = N // BN

    def body(o0_ref, agg_ref, g_ref, be_ref, w_ref, root_ref, b_ref,
             h_ref, o_ref):
        r = pl.program_id(1)
        s = o0_ref[...] + agg_ref[0] + agg_ref[1]
        act = _gelu(_ln(s, g_ref[...], be_ref[...]))
        h_ref[0] = jnp.dot(act, w_ref[0], preferred_element_type=jnp.float32)

        @pl.when(r == 0)
        def _():
            o_ref[...] = jnp.dot(act, root_ref[...],
                                 preferred_element_type=jnp.float32) \
                         + b_ref[...]

    return pl.pallas_call(
        body,
        grid=(NB, R),
        in_specs=[
            pl.BlockSpec((BN, H), lambda i, r: (i, 0)),
            pl.BlockSpec((2, BN, H), lambda i, r: (0, i, 0)),
            pl.BlockSpec((1, H), lambda i, r: (0, 0)),
            pl.BlockSpec((1, H), lambda i, r: (0, 0)),
            pl.BlockSpec((1, H, D), lambda i, r: (r, 0, 0)),
            pl.BlockSpec((H, D), lambda i, r: (0, 0)),
            pl.BlockSpec((1, D), lambda i, r: (0, 0)),
        ],
        out_specs=[
            pl.BlockSpec((1, BN, D), lambda i, r: (r, i, 0)),
            pl.BlockSpec((BN, D), lambda i, r: (i, 0)),
        ],
        out_shape=[
            jax.ShapeDtypeStruct((R, N, D), jnp.float32),
            jax.ShapeDtypeStruct((N, D), jnp.float32),
        ],
    )(out0, agg, g1r, be1r, W, root, b2r)


def _tc_final(x, out0, agg, g2r, be2r, BN):
    N, D = x.shape
    NB = N // BN

    def body(x_ref, o0_ref, agg_ref, g_ref, be_ref, out_ref):
        h = _ln(o0_ref[...] + agg_ref[0] + agg_ref[1], g_ref[...], be_ref[...])
        out = x_ref[...] + h
        nrm = jnp.sqrt(jnp.sum(out * out, axis=-1, keepdims=True))
        out_ref[...] = out / jnp.maximum(nrm, 1e-12)

    return pl.pallas_call(
        body,
        grid=(NB,),
        in_specs=[
            pl.BlockSpec((BN, D), lambda i: (i, 0)),
            pl.BlockSpec((BN, D), lambda i: (i, 0)),
            pl.BlockSpec((2, BN, D), lambda i: (0, i, 0)),
            pl.BlockSpec((1, D), lambda i: (0, 0)),
            pl.BlockSpec((1, D), lambda i: (0, 0)),
        ],
        out_specs=pl.BlockSpec((BN, D), lambda i: (i, 0)),
        out_shape=jax.ShapeDtypeStruct((N, D), jnp.float32),
    )(x, out0, agg, g2r, be2r)


_CH = 80


def _sc_prep(et, src, dst, N, R):
    E = et.shape[0]
    RN = R * N
    epw = E // NS
    nch = epw // _CH
    zcs = RN // NS
    ZB = 2000
    assert epw % _CH == 0 and zcs % ZB == 0

    mesh = plsc.VectorSubcoreMesh(**_SC_MESH)

    @functools.partial(
        pl.kernel, mesh=mesh,
        out_type=[
            jax.ShapeDtypeStruct((E,), jnp.int32),
            jax.ShapeDtypeStruct((E,), jnp.float32),
        ],
        scratch_types=[
            pltpu.VMEM((_CH,), jnp.int32),
            pltpu.VMEM((_CH,), jnp.int32),
            pltpu.VMEM((_CH,), jnp.int32),
            pltpu.VMEM((_CH,), jnp.int32),
            pltpu.VMEM((_CH,), jnp.int32),
            pltpu.VMEM((_CH,), jnp.float32),
            pltpu.VMEM((ZB,), jnp.float32),
            pltpu.VMEM_SHARED((RN,), jnp.float32),
        ],
    )
    def body(et_h, src_h, dst_h, gidx_h, w_h,
             et_v, src_v, dst_v, gidx_v, cidx_v, w_v, zb_v, cnt_sh):
        core = lax.axis_index("c")
        sub = lax.axis_index("s")

        @pl.when(core == 0)
        def _():
            def zb_zero(k, _):
                zb_v[pl.ds(k * 16, 16)] = jnp.zeros((16,), jnp.float32)
                return 0
            lax.fori_loop(0, ZB // 16, zb_zero, 0)
            for z in range(zcs // ZB):
                pltpu.sync_copy(zb_v, cnt_sh.at[pl.ds(sub * zcs + z * ZB, ZB)])
            def ones_fill(k, _):
                w_v[pl.ds(k * 16, 16)] = jnp.ones((16,), jnp.float32)
                return 0
            lax.fori_loop(0, _CH // 16, ones_fill, 0)
            plsc.subcore_barrier()

            def p1(c, _):
                base = sub * epw + c * _CH
                pltpu.sync_copy(et_h.at[pl.ds(base, _CH)], et_v)
                pltpu.sync_copy(src_h.at[pl.ds(base, _CH)], src_v)
                pltpu.sync_copy(dst_h.at[pl.ds(base, _CH)], dst_v)

                def ix(k, _):
                    sl = pl.ds(k * 16, 16)
                    e16 = et_v[sl] * N
                    gidx_v[sl] = e16 + src_v[sl]
                    cidx_v[sl] = e16 + dst_v[sl]
                    return 0
                lax.fori_loop(0, _CH // 16, ix, 0)
                pltpu.sync_copy(gidx_v, gidx_h.at[pl.ds(base, _CH)])
                pltpu.sync_copy(w_v, cnt_sh.at[cidx_v], add=True)
                return 0
            lax.fori_loop(0, nch, p1, 0)
            plsc.subcore_barrier()

            def inv(z, _):
                off = sub * zcs + z * ZB
                pltpu.sync_copy(cnt_sh.at[pl.ds(off, ZB)], zb_v)

                def iv(k, _):
                    sl = pl.ds(k * 16, 16)
                    zb_v[sl] = 1.0 / jnp.maximum(zb_v[sl], 1.0)
                    return 0
                lax.fori_loop(0, ZB // 16, iv, 0)
                pltpu.sync_copy(zb_v, cnt_sh.at[pl.ds(off, ZB)])
                return 0
            lax.fori_loop(0, zcs // ZB, inv, 0)
            plsc.subcore_barrier()

            def p2(c, _):
                base = sub * epw + c * _CH
                pltpu.sync_copy(et_h.at[pl.ds(base, _CH)], et_v)
                pltpu.sync_copy(dst_h.at[pl.ds(base, _CH)], dst_v)

                def ix(k, _):
                    sl = pl.ds(k * 16, 16)
                    cidx_v[sl] = et_v[sl] * N + dst_v[sl]
                    return 0
                lax.fori_loop(0, _CH // 16, ix, 0)
                pltpu.sync_copy(cnt_sh.at[cidx_v], w_v)
                pltpu.sync_copy(w_v, w_h.at[pl.ds(base, _CH)])
                return 0
            lax.fori_loop(0, nch, p2, 0)

    return body(et, src, dst)


def _sc_agg(h2d, gidx, dst, w, N):
    E = gidx.shape[0]
    H = h2d.shape[1]
    epw = E // NW
    nch = epw // _CH
    ZR = 200
    nzc = N // ZR
    nzi = (nzc + NS - 1) // NS
    assert epw % _CH == 0 and N % ZR == 0 and ZR % 8 == 0

    mesh = plsc.VectorSubcoreMesh(**_SC_MESH)

    @functools.partial(
        pl.kernel, mesh=mesh,
        out_type=jax.ShapeDtypeStruct((NC, N, H), jnp.float32),
        scratch_types=[
            pltpu.VMEM((_CH,), jnp.int32),
            pltpu.VMEM((_CH,), jnp.int32),
            pltpu.VMEM((_CH,), jnp.float32),
            pltpu.VMEM((_CH, 128), jnp.float32),
            pltpu.VMEM((ZR, 128), jnp.float32),
            pltpu.VMEM_SHARED((N, 128), jnp.float32),
        ],
    )
    def body(h_h, gidx_h, dst_h, w_h, agg_h,
             gidx_v, dst_v, w_v, rows_v, zb_v, acc_sh):
        core = lax.axis_index("c")
        sub = lax.axis_index("s")
        wid = sub * NC + core

        def zb_zero(j, _):
            for k in range(8):
                zb_v[j, pl.ds(k * 16, 16)] = jnp.zeros((16,), jnp.float32)
            return 0
        lax.fori_loop(0, ZR, zb_zero, 0)
        for z in range(nzi):
            zc = sub + z * NS

            @pl.when(zc < nzc)
            def _():
                pltpu.sync_copy(zb_v, acc_sh.at[pl.ds(zc * ZR, ZR)])
        plsc.subcore_barrier()

        def chunk(c, _):
            base = wid * epw + c * _CH
            pltpu.sync_copy(gidx_h.at[pl.ds(base, _CH)], gidx_v)
            pltpu.sync_copy(dst_h.at[pl.ds(base, _CH)], dst_v)
            pltpu.sync_copy(w_h.at[pl.ds(base, _CH)], w_v)
            pltpu.sync_copy(h_h.at[gidx_v], rows_v)

            def scale(jj, _):
                v16w = w_v[pl.ds(jj * 16, 16)]
                for lane in range(16):
                    wv = jnp.full((16,), v16w[lane])
                    j = jj * 16 + lane
                    for k in range(8):
                        sl = pl.ds(k * 16, 16)
                        rows_v[j, sl] = rows_v[j, sl] * wv
                return 0
            lax.fori_loop(0, _CH // 16, scale, 0)
            pltpu.sync_copy(rows_v, acc_sh.at[dst_v], add=True)
            return 0
        lax.fori_loop(0, nch, chunk, 0)
        plsc.subcore_barrier()

        for z in range(nzi):
            zc = sub + z * NS

            @pl.when(zc < nzc)
            def _():
                sl = pl.ds(zc * ZR, ZR)
                pltpu.sync_copy(acc_sh.at[sl], agg_h.at[core].at[sl])

    return body(h2d, gidx, dst, w)


def kernel(x, edge_index, edge_type, node_type, node_type_emb,
           W1, root1, b1, g1, be1, W2, root2, b2, g2, be2):
    N, D = x.shape
    R, _, H = W1.shape
    BN = 2000

    src = edge_index[0].astype(jnp.int32)
    dst = edge_index[1].astype(jnp.int32)
    et = edge_type.astype(jnp.int32)
    nt2 = node_type.reshape(N, 1).astype(jnp.int32)

    h1, out0_1 = _tc_first(x, nt2, node_type_emb, W1, root1,
                           b1.reshape(1, H), BN)
    gidx, w_e = _sc_prep(et, src, dst, N, R)
    agg1 = _sc_agg(h1.reshape(R * N, H), gidx, dst, w_e, N)
    h2, out0_2 = _tc_mid(out0_1, agg1, g1.reshape(1, H), be1.reshape(1, H),
                         W2, root2, b2.reshape(1, D), BN)
    agg2 = _sc_agg(h2.reshape(R * N, D), gidx, dst, w_e, N)
    return _tc_final(x, out0_2, agg2, g2.reshape(1, D), be2.reshape(1, D), BN)
